# baseline (speedup 1.0000x reference)
import functools

import jax
import jax.numpy as jnp
from jax import lax
from jax.experimental import pallas as pl
from jax.experimental.pallas import tpu as pltpu
from jax.experimental.pallas import tpu_sc as plsc

M = 4096
K = 32
N = 8192
R = 256
NB = M // R
COMMITMENT_COST = 0.25

NC, NS, L = 2, 16, 16
NW = NC * NS
TROWS = M // NW
ZCH = 8
ZREP = TROWS // ZCH
W128 = 128
G128 = N // W128

_sc_mesh = plsc.VectorSubcoreMesh(core_axis_name="c", subcore_axis_name="s")


def _argmin_body(x_ref, xsq_ref, wsq_ref, iota_ref, w_ref, idx_ref):
    x = x_ref[...]
    xm2 = x * (-2.0)
    w = w_ref[...]
    xwm2 = lax.dot_general(xm2, w, (((1,), (1,)), ((), ())),
                           preferred_element_type=jnp.float32)
    d = (xsq_ref[...] + wsq_ref[...]) + xwm2
    mval = jnp.min(d, axis=1, keepdims=True)
    iota_f = jnp.broadcast_to(iota_ref[...], (R, N))
    idx_f = jnp.min(jnp.where(d == mval, iota_f, jnp.float32(N)), axis=1)
    idx_ref[0, 0, :] = idx_f.astype(jnp.int32)


@jax.jit
def _argmin_call(x_flat, xsq, wsq, iota_row, embedding_weight):
    return pl.pallas_call(
        _argmin_body,
        grid=(NB,),
        in_specs=[
            pl.BlockSpec((R, K), lambda i: (i, 0)),
            pl.BlockSpec((R, 1), lambda i: (i, 0)),
            pl.BlockSpec((1, N), lambda i: (0, 0)),
            pl.BlockSpec((1, N), lambda i: (0, 0)),
            pl.BlockSpec((N, K), lambda i: (0, 0)),
        ],
        out_specs=pl.BlockSpec((1, 1, R), lambda i: (i, 0, 0)),
        out_shape=jax.ShapeDtypeStruct((NB, 1, R), jnp.int32),
    )(x_flat, xsq, wsq, iota_row, embedding_weight)


@functools.partial(
    pl.kernel,
    out_type=jax.ShapeDtypeStruct((M, N), jnp.float32),
    mesh=_sc_mesh,
    scratch_types=[
        pltpu.VMEM((ZCH, N), jnp.float32),
        pltpu.SemaphoreType.DMA,
    ],
)
def _sc_zero(enc_hbm, zbuf, sem):
    wid = lax.axis_index("s") * NC + lax.axis_index("c")
    base = wid * TROWS

    def _zero_row(r, _):
        def _zero16(i, _i):
            off = pl.multiple_of(i * L, L)
            zbuf[r, pl.ds(off, L)] = jnp.zeros((L,), jnp.float32)
            return 0

        lax.fori_loop(0, N // L, _zero16, 0)
        return 0

    lax.fori_loop(0, ZCH, _zero_row, 0)

    copies = [
        pltpu.async_copy(zbuf, enc_hbm.at[pl.ds(base + ci * ZCH, ZCH), :], sem)
        for ci in range(ZREP)
    ]
    for cp in copies:
        cp.wait()


@functools.partial(
    pl.kernel,
    out_type=(
        jax.ShapeDtypeStruct((M, W128), jnp.float32),
        jax.ShapeDtypeStruct((NC, G128, W128), jnp.float32),
    ),
    mesh=_sc_mesh,
    scratch_types=[
        pltpu.VMEM((TROWS,), jnp.int32),
        pltpu.VMEM((TROWS,), jnp.int32),
        pltpu.VMEM((TROWS,), jnp.int32),
        pltpu.VMEM((TROWS,), jnp.int32),
        pltpu.VMEM((TROWS, W128), jnp.float32),
        pltpu.VMEM((TROWS, W128), jnp.float32),
        pltpu.VMEM((G128, W128), jnp.float32),
        pltpu.VMEM_SHARED((G128, W128), jnp.float32),
        pltpu.SemaphoreType.DMA,
    ],
)
def _sc_scatter(idx_hbm, eye_hbm, wpad_hbm, enc_flat_ref, q_hbm, cnt_hbm,
                idx_v, col_v, offs_v, hoffs_v, src_v, rows_v, bins_v,
                shared_bins, sem):
    sid = lax.axis_index("s")
    cid = lax.axis_index("c")
    wid = sid * NC + cid
    base = wid * TROWS

    pltpu.sync_copy(idx_hbm.at[pl.ds(base, TROWS)], idx_v)

    @pl.when(sid == 0)
    def _init_bins():
        def _zero_bins(r, _):
            def _zero16(i, _i):
                off = pl.multiple_of(i * L, L)
                bins_v[r, pl.ds(off, L)] = jnp.zeros((L,), jnp.float32)
                return 0

            lax.fori_loop(0, W128 // L, _zero16, 0)
            return 0

        lax.fori_loop(0, G128, _zero_bins, 0)
        pltpu.sync_copy(bins_v, shared_bins)

    lane = lax.broadcasted_iota(jnp.int32, (L,), 0)
    for g in range(TROWS // L):
        v = idx_v[pl.ds(g * L, L)]
        col_v[pl.ds(g * L, L)] = jnp.bitwise_and(v, W128 - 1)
        hoffs_v[pl.ds(g * L, L)] = lax.shift_right_logical(v, 7)
        offs_v[pl.ds(g * L, L)] = (
            (base + g * L + lane) * G128 + lax.shift_right_logical(v, 7))

    pltpu.async_copy(eye_hbm.at[col_v], src_v, sem).wait()
    pltpu.async_copy(src_v, enc_flat_ref.at[offs_v], sem).wait()
    pltpu.async_copy(wpad_hbm.at[idx_v], rows_v, sem).wait()
    pltpu.sync_copy(rows_v, q_hbm.at[pl.ds(base, TROWS), :])
    plsc.subcore_barrier()
    pltpu.sync_copy(src_v, shared_bins.at[hoffs_v], add=True)
    plsc.subcore_barrier()

    @pl.when(sid == 0)
    def _emit_bins():
        pltpu.sync_copy(shared_bins, cnt_hbm.at[cid])


def _finish_body(q_ref, x_ref, cnt_ref, qst_ref, loss_ref, perp_ref):
    q = q_ref[...][:, :K]
    x = x_ref[...]
    qst_ref[...] = x + (q - x)
    loss_ref[0, 0] = COMMITMENT_COST * jnp.sum((q - x) ** 2) / (M * K)
    counts = jnp.sum(cnt_ref[...], axis=0)
    p = counts * (1.0 / M)
    perp_ref[0, 0] = jnp.exp(-jnp.sum(p * jnp.log(p + 1e-10)))


@jax.jit
def _finish_call(q, x_flat, counts):
    return pl.pallas_call(
        _finish_body,
        in_specs=[
            pl.BlockSpec((M, W128), lambda: (0, 0)),
            pl.BlockSpec((M, K), lambda: (0, 0)),
            pl.BlockSpec((NC, N), lambda: (0, 0)),
        ],
        out_specs=(
            pl.BlockSpec((M, K), lambda: (0, 0)),
            pl.BlockSpec(memory_space=pltpu.SMEM),
            pl.BlockSpec(memory_space=pltpu.SMEM),
        ),
        out_shape=(
            jax.ShapeDtypeStruct((M, K), jnp.float32),
            jax.ShapeDtypeStruct((1, 1), jnp.float32),
            jax.ShapeDtypeStruct((1, 1), jnp.float32),
        ),
    )(q, x_flat, counts)


@jax.jit
def kernel(inputs, embedding_weight):
    x_flat = jnp.transpose(inputs, (0, 2, 3, 1)).reshape(M, K)
    xsq = jnp.sum(x_flat ** 2, axis=1, keepdims=True)
    wsq = jnp.sum(embedding_weight ** 2, axis=1).reshape(1, N)
    iota_row = jnp.arange(N, dtype=jnp.float32).reshape(1, N)

    idx3 = _argmin_call(x_flat, xsq, wsq, iota_row, embedding_weight)
    idx = idx3.reshape(M)

    enc_zero = _sc_zero()
    enc_ref = jax.new_ref(enc_zero.reshape(M * G128, W128))
    eye128 = jnp.eye(W128, dtype=jnp.float32)
    wpad = jnp.pad(embedding_weight, ((0, 0), (0, W128 - K)))
    q, counts = _sc_scatter(idx, eye128, wpad, enc_ref)
    enc = enc_ref[...].reshape(M, N)

    qst, loss, perp = _finish_call(q, x_flat, counts.reshape(NC, N))
    quantized_out = jnp.transpose(qst.reshape(16, 16, 16, K), (0, 3, 1, 2))
    return (quantized_out,
            loss.reshape(()),
            perp.reshape(()),
            idx.reshape(M, 1),
            enc)


# speedup vs baseline: 3.4441x; 3.4441x over previous
import functools

import jax
import jax.numpy as jnp
from jax import lax
from jax.experimental import pallas as pl
from jax.experimental.pallas import tpu as pltpu

M = 4096
K = 32
N = 8192
R = 512
NB = M // R
COMMITMENT_COST = 0.25


def _vq_body(x_ref, xsq_ref, wsq_ref, iota_ref, w_ref, enc_ref, q_ref,
             idx_ref, loss_ref, perp_ref, counts_ref, acc_ref):
    step = pl.program_id(0)

    @pl.when(step == 0)
    def _init():
        counts_ref[...] = jnp.zeros_like(counts_ref)
        acc_ref[0] = 0.0

    x = x_ref[...]
    xm2 = x * (-2.0)
    w = w_ref[...]
    xwm2 = lax.dot_general(xm2, w, (((1,), (1,)), ((), ())),
                           preferred_element_type=jnp.float32)
    d = (xsq_ref[...] + wsq_ref[...]) + xwm2
    mval = jnp.min(d, axis=1, keepdims=True)
    d_i = lax.bitcast_convert_type(d, jnp.int32)
    mval_i = lax.bitcast_convert_type(mval, jnp.int32)
    iota_f = jnp.broadcast_to(iota_ref[...], (R, N))
    idx_f = jnp.min(jnp.where(d_i == mval_i, iota_f, jnp.float32(N)), axis=1)
    idx_ref[0, 0, :] = idx_f.astype(jnp.int32)

    iota_i = lax.bitcast_convert_type(iota_f, jnp.int32)
    idx_i = lax.bitcast_convert_type(idx_f, jnp.int32)
    enc = (iota_i == idx_i[:, None]).astype(jnp.float32)
    enc_ref[...] = enc

    q = lax.dot_general(enc, w, (((1,), (0,)), ((), ())),
                        preferred_element_type=jnp.float32)
    q_ref[...] = x + (q - x)

    ones_r = jnp.ones((1, R), jnp.float32)
    counts_ref[...] += lax.dot_general(ones_r, enc, (((1,), (0,)), ((), ())),
                                       preferred_element_type=jnp.float32)
    acc_ref[0] += jnp.sum((q - x) ** 2)

    @pl.when(step == NB - 1)
    def _fini():
        loss_ref[0, 0] = COMMITMENT_COST * acc_ref[0] / (M * K)
        p = counts_ref[...] * (1.0 / M)
        perp_ref[0, 0] = jnp.exp(-jnp.sum(p * jnp.log(p + 1e-10)))


@functools.partial(jax.jit, static_argnames=("interpret",))
def _vq_call(x_flat, xsq, wsq, iota_row, embedding_weight, interpret=False):
    out_shapes = (
        jax.ShapeDtypeStruct((M, N), jnp.float32),
        jax.ShapeDtypeStruct((M, K), jnp.float32),
        jax.ShapeDtypeStruct((NB, 1, R), jnp.int32),
        jax.ShapeDtypeStruct((1, 1), jnp.float32),
        jax.ShapeDtypeStruct((1, 1), jnp.float32),
    )
    out_specs = (
        pl.BlockSpec((R, N), lambda i: (i, 0)),
        pl.BlockSpec((R, K), lambda i: (i, 0)),
        pl.BlockSpec((1, 1, R), lambda i: (i, 0, 0)),
        pl.BlockSpec(memory_space=pltpu.SMEM),
        pl.BlockSpec(memory_space=pltpu.SMEM),
    )
    in_specs = [
        pl.BlockSpec((R, K), lambda i: (i, 0)),
        pl.BlockSpec((R, 1), lambda i: (i, 0)),
        pl.BlockSpec((1, N), lambda i: (0, 0)),
        pl.BlockSpec((1, N), lambda i: (0, 0)),
        pl.BlockSpec((N, K), lambda i: (0, 0)),
    ]
    return pl.pallas_call(
        _vq_body,
        grid=(NB,),
        in_specs=in_specs,
        out_specs=out_specs,
        out_shape=out_shapes,
        scratch_shapes=[
            pltpu.VMEM((1, N), jnp.float32),
            pltpu.SMEM((1,), jnp.float32),
        ],
        compiler_params=pltpu.CompilerParams(vmem_limit_bytes=128 * 1024 * 1024),
        interpret=interpret,
    )(x_flat, xsq, wsq, iota_row, embedding_weight)


def kernel(inputs, embedding_weight, interpret=False):
    x_flat = jnp.transpose(inputs, (0, 2, 3, 1)).reshape(M, K)
    xsq = jnp.sum(x_flat ** 2, axis=1, keepdims=True)
    wsq = jnp.sum(embedding_weight ** 2, axis=1).reshape(1, N)
    iota_row = jnp.arange(N, dtype=jnp.float32).reshape(1, N)
    enc, q, idx, loss, perp = _vq_call(x_flat, xsq, wsq, iota_row,
                                       embedding_weight, interpret=interpret)
    quantized_out = jnp.transpose(q.reshape(16, 16, 16, K), (0, 3, 1, 2))
    return (quantized_out,
            loss.reshape(()),
            perp.reshape(()),
            idx.reshape(M, 1),
            enc)
